# baseline (speedup 1.0000x reference)
import functools

import jax
import jax.numpy as jnp
from jax import lax
from jax.experimental import pallas as pl
from jax.experimental.pallas import tpu as pltpu
from jax.experimental.pallas import tpu_sc as plsc

_NW = 8


def _sc_gather(table, idx):
    n, emb_dim = table.shape
    b = idx.shape[0]
    b_per_w = b // _NW
    mesh = plsc.VectorSubcoreMesh(core_axis_name="c", subcore_axis_name="s")

    @functools.partial(
        pl.kernel,
        mesh=mesh,
        out_type=jax.ShapeDtypeStruct((b, emb_dim), jnp.float32),
        scratch_types=[
            pltpu.VMEM((b_per_w,), jnp.int32),
            pltpu.VMEM((b_per_w, emb_dim), jnp.float32),
            pltpu.SemaphoreType.DMA,
        ],
    )
    def gather_kernel(table_hbm, idx_hbm, out_hbm, idx_v, rows_v, sem):
        wid = lax.axis_index("s") * 2 + lax.axis_index("c")

        @pl.when(wid < _NW)
        def _():
            base = wid * b_per_w
            pltpu.sync_copy(idx_hbm.at[pl.ds(base, b_per_w)], idx_v)
            pltpu.async_copy(table_hbm.at[idx_v], rows_v, sem).wait()
            pltpu.sync_copy(rows_v, out_hbm.at[pl.ds(base, b_per_w)])

    return gather_kernel(table, idx)


def _film_body(emb_ref, x_ref, o_ref):
    dim = x_ref.shape[1]
    w = emb_ref[0, 0, :dim]
    bias = emb_ref[0, 0, dim:]
    o_ref[0] = x_ref[0] * w[:, None, None] + bias[:, None, None]


def _tc_film(imgs, emb3):
    b, dim, h, w = imgs.shape
    return pl.pallas_call(
        _film_body,
        grid=(b,),
        in_specs=[
            pl.BlockSpec((1, 1, 2 * dim), lambda i: (i, 0, 0)),
            pl.BlockSpec((1, dim, h, w), lambda i: (i, 0, 0, 0)),
        ],
        out_specs=pl.BlockSpec((1, dim, h, w), lambda i: (i, 0, 0, 0)),
        out_shape=jax.ShapeDtypeStruct((b, dim, h, w), jnp.float32),
    )(emb3, imgs)


@jax.jit
def kernel(imgs, ys, proj_weight):
    b, dim, h, w = imgs.shape
    emb = _sc_gather(proj_weight, ys.astype(jnp.int32))
    return _tc_film(imgs, emb.reshape(b, 1, 2 * dim))


# speedup vs baseline: 10.1645x; 10.1645x over previous
import functools

import jax
import jax.numpy as jnp
from jax import lax
from jax.experimental import pallas as pl
from jax.experimental.pallas import tpu as pltpu
from jax.experimental.pallas import tpu_sc as plsc

_NW = 8


def _sc_gather(table, idx):
    n, emb_dim = table.shape
    b = idx.shape[0]
    b_per_w = b // _NW
    mesh = plsc.VectorSubcoreMesh(core_axis_name="c", subcore_axis_name="s")

    @functools.partial(
        pl.kernel,
        mesh=mesh,
        out_type=jax.ShapeDtypeStruct((b, emb_dim), jnp.float32),
        scratch_types=[
            pltpu.VMEM((b_per_w,), jnp.int32),
            pltpu.VMEM((b_per_w, emb_dim), jnp.float32),
            pltpu.SemaphoreType.DMA,
        ],
    )
    def gather_kernel(table_hbm, idx_hbm, out_hbm, idx_v, rows_v, sem):
        wid = lax.axis_index("s") * 2 + lax.axis_index("c")

        @pl.when(wid < _NW)
        def _():
            base = wid * b_per_w
            pltpu.sync_copy(idx_hbm.at[pl.ds(base, b_per_w)], idx_v)
            pltpu.async_copy(table_hbm.at[idx_v], rows_v, sem).wait()
            pltpu.sync_copy(rows_v, out_hbm.at[pl.ds(base, b_per_w)])

    return gather_kernel(table, idx)


_BB = 4


def _film_body(emb_ref, x_ref, o_ref):
    dim = x_ref.shape[-1]
    w = emb_ref[:, 0, :dim]
    bias = emb_ref[:, 0, dim:]
    o_ref[...] = x_ref[...] * w[:, None, None, :] + bias[:, None, None, :]


def _tc_film(imgs_t, emb3):
    b, h, w, dim = imgs_t.shape
    return pl.pallas_call(
        _film_body,
        grid=(b // _BB,),
        in_specs=[
            pl.BlockSpec((_BB, 1, 2 * dim), lambda i: (i, 0, 0)),
            pl.BlockSpec((_BB, h, w, dim), lambda i: (i, 0, 0, 0)),
        ],
        out_specs=pl.BlockSpec((_BB, h, w, dim), lambda i: (i, 0, 0, 0)),
        out_shape=jax.ShapeDtypeStruct((b, h, w, dim), jnp.float32),
    )(emb3, imgs_t)


@jax.jit
def kernel(imgs, ys, proj_weight):
    b, dim, h, w = imgs.shape
    emb = _sc_gather(proj_weight, ys.astype(jnp.int32))
    out_t = _tc_film(imgs.transpose(0, 2, 3, 1), emb.reshape(b, 1, 2 * dim))
    return out_t.transpose(0, 3, 1, 2)


# speedup vs baseline: 10.9194x; 1.0743x over previous
import functools

import jax
import jax.numpy as jnp
from jax import lax
from jax.experimental import pallas as pl
from jax.experimental.pallas import tpu as pltpu
from jax.experimental.pallas import tpu_sc as plsc

_NW = 8


def _sc_gather(table, idx):
    n, emb_dim = table.shape
    b = idx.shape[0]
    b_per_w = b // _NW
    mesh = plsc.VectorSubcoreMesh(
        core_axis_name="c", subcore_axis_name="s", num_cores=1
    )

    @functools.partial(
        pl.kernel,
        mesh=mesh,
        out_type=jax.ShapeDtypeStruct((b, emb_dim), jnp.float32),
        scratch_types=[
            pltpu.VMEM((b_per_w,), jnp.int32),
            pltpu.VMEM((b_per_w, emb_dim), jnp.float32),
            pltpu.SemaphoreType.DMA,
        ],
    )
    def gather_kernel(table_hbm, idx_hbm, out_hbm, idx_v, rows_v, sem):
        wid = lax.axis_index("s")

        @pl.when(wid < _NW)
        def _():
            base = wid * b_per_w
            pltpu.sync_copy(idx_hbm.at[pl.ds(base, b_per_w)], idx_v)
            pltpu.async_copy(table_hbm.at[idx_v], rows_v, sem).wait()
            pltpu.sync_copy(rows_v, out_hbm.at[pl.ds(base, b_per_w)])

    return gather_kernel(table, idx)


_BB = 16
_DB = 128
_HEAD = 16
_HBB = 4


def _film_head_body(ys_ref, pw_ref, x_ref, o_ref):
    i = pl.program_id(0)
    dim = x_ref.shape[-1]
    rows = [
        pw_ref[pl.ds(ys_ref[i * _HBB + k], 1), :] for k in range(_HBB)
    ]
    wb = jnp.concatenate(rows, axis=0)
    w = wb[:, :dim]
    bias = wb[:, dim:]
    o_ref[...] = x_ref[...] * w[:, None, None, :] + bias[:, None, None, :]


def _tc_film_head(imgs_t, ys, table):
    b, h, w, dim = imgs_t.shape
    n, emb_dim = table.shape
    return pl.pallas_call(
        _film_head_body,
        grid=(_HEAD // _HBB,),
        in_specs=[
            pl.BlockSpec(memory_space=pltpu.SMEM),
            pl.BlockSpec((n, emb_dim), lambda i: (0, 0)),
            pl.BlockSpec((_HBB, h, w, dim), lambda i: (i, 0, 0, 0)),
        ],
        out_specs=pl.BlockSpec((_HBB, h, w, dim), lambda i: (i, 0, 0, 0)),
        out_shape=jax.ShapeDtypeStruct((b, h, w, dim), jnp.float32),
    )(ys, table, imgs_t)


def _film_main_body(w_ref, b_ref, x_ref, part_ref, o_ref):
    del part_ref
    o_ref[...] = (
        x_ref[...] * w_ref[...][:, None, None, :] + b_ref[...][:, None, None, :]
    )


def _tc_film_main(imgs_t, emb, partial_out):
    b, h, w, dim = imgs_t.shape
    nj = dim // _DB
    i0 = _HEAD // _BB
    return pl.pallas_call(
        _film_main_body,
        grid=((b - _HEAD) // _BB, nj),
        in_specs=[
            pl.BlockSpec((_BB, _DB), lambda i, j: (i + i0, j)),
            pl.BlockSpec((_BB, _DB), lambda i, j: (i + i0, j + nj)),
            pl.BlockSpec((_BB, h, w, _DB), lambda i, j: (i + i0, 0, 0, j)),
            pl.BlockSpec(memory_space=pl.ANY),
        ],
        out_specs=pl.BlockSpec((_BB, h, w, _DB), lambda i, j: (i + i0, 0, 0, j)),
        out_shape=jax.ShapeDtypeStruct((b, h, w, dim), jnp.float32),
        input_output_aliases={3: 0},
    )(emb, emb, imgs_t, partial_out)


@jax.jit
def kernel(imgs, ys, proj_weight):
    b, dim, h, w = imgs.shape
    ys32 = ys.astype(jnp.int32)
    imgs_t = imgs.transpose(0, 2, 3, 1)
    emb = _sc_gather(proj_weight, ys32)
    partial_out = _tc_film_head(imgs_t, ys32, proj_weight)
    out_t = _tc_film_main(imgs_t, emb, partial_out)
    return out_t.transpose(0, 3, 1, 2)


# speedup vs baseline: 10.9304x; 1.0010x over previous
import functools

import jax
import jax.numpy as jnp
from jax import lax
from jax.experimental import pallas as pl
from jax.experimental.pallas import tpu as pltpu
from jax.experimental.pallas import tpu_sc as plsc

_NW = 8


def _sc_gather(table, idx):
    n, emb_dim = table.shape
    b = idx.shape[0]
    b_per_w = b // _NW
    mesh = plsc.VectorSubcoreMesh(
        core_axis_name="c", subcore_axis_name="s", num_cores=1
    )

    @functools.partial(
        pl.kernel,
        mesh=mesh,
        out_type=jax.ShapeDtypeStruct((b, emb_dim), jnp.float32),
        scratch_types=[
            pltpu.VMEM((b_per_w,), jnp.int32),
            pltpu.VMEM((b_per_w, emb_dim), jnp.float32),
            pltpu.SemaphoreType.DMA,
        ],
    )
    def gather_kernel(table_hbm, idx_hbm, out_hbm, idx_v, rows_v, sem):
        wid = lax.axis_index("s")

        @pl.when(wid < _NW)
        def _():
            base = wid * b_per_w
            pltpu.sync_copy(idx_hbm.at[pl.ds(base, b_per_w)], idx_v)
            pltpu.async_copy(table_hbm.at[idx_v], rows_v, sem).wait()
            pltpu.sync_copy(rows_v, out_hbm.at[pl.ds(base, b_per_w)])

    return gather_kernel(table, idx)


_BB = 16
_DB = 128
_HEAD = 16
_H1 = 8
_HBB = 4


def _make_film_head_body(start):
    def body(ys_ref, pw_ref, x_ref, *rest):
        o_ref = rest[-1]
        i = pl.program_id(0)
        dim = x_ref.shape[-1]
        rows = [
            pw_ref[pl.ds(ys_ref[start + i * _HBB + k], 1), :]
            for k in range(_HBB)
        ]
        wb = jnp.concatenate(rows, axis=0)
        w = wb[:, :dim]
        bias = wb[:, dim:]
        o_ref[...] = x_ref[...] * w[:, None, None, :] + bias[:, None, None, :]

    return body


def _tc_film_head(imgs_t, ys, table, start, count, partial_out=None):
    b, h, w, dim = imgs_t.shape
    n, emb_dim = table.shape
    i0 = start // _HBB
    in_specs = [
        pl.BlockSpec(memory_space=pltpu.SMEM),
        pl.BlockSpec((n, emb_dim), lambda i: (0, 0)),
        pl.BlockSpec((_HBB, h, w, dim), lambda i: (i + i0, 0, 0, 0)),
    ]
    args = [ys, table, imgs_t]
    aliases = {}
    if partial_out is not None:
        in_specs.append(pl.BlockSpec(memory_space=pl.ANY))
        args.append(partial_out)
        aliases = {3: 0}
    return pl.pallas_call(
        _make_film_head_body(start),
        grid=(count // _HBB,),
        in_specs=in_specs,
        out_specs=pl.BlockSpec((_HBB, h, w, dim), lambda i: (i + i0, 0, 0, 0)),
        out_shape=jax.ShapeDtypeStruct((b, h, w, dim), jnp.float32),
        input_output_aliases=aliases,
    )(*args)


def _film_main_body(w_ref, b_ref, x_ref, part_ref, o_ref):
    del part_ref
    o_ref[...] = (
        x_ref[...] * w_ref[...][:, None, None, :] + b_ref[...][:, None, None, :]
    )


def _film_all(imgs_t, ys32, proj_weight):
    part1 = _tc_film_head(imgs_t, ys32, proj_weight, 0, _H1)
    ys_dep = lax.optimization_barrier((ys32, part1))[0]
    emb = _sc_gather(proj_weight, ys_dep)
    part2 = _tc_film_head(
        imgs_t, ys32, proj_weight, _H1, _HEAD - _H1, partial_out=part1
    )
    return _tc_film_main(imgs_t, emb, part2)


def _tc_film_main(imgs_t, emb, partial_out):
    b, h, w, dim = imgs_t.shape
    nj = dim // _DB
    i0 = _HEAD // _BB
    return pl.pallas_call(
        _film_main_body,
        grid=((b - _HEAD) // _BB, nj),
        in_specs=[
            pl.BlockSpec((_BB, _DB), lambda i, j: (i + i0, j)),
            pl.BlockSpec((_BB, _DB), lambda i, j: (i + i0, j + nj)),
            pl.BlockSpec((_BB, h, w, _DB), lambda i, j: (i + i0, 0, 0, j)),
            pl.BlockSpec(memory_space=pl.ANY),
        ],
        out_specs=pl.BlockSpec((_BB, h, w, _DB), lambda i, j: (i + i0, 0, 0, j)),
        out_shape=jax.ShapeDtypeStruct((b, h, w, dim), jnp.float32),
        input_output_aliases={3: 0},
    )(emb, emb, imgs_t, partial_out)


@jax.jit
def kernel(imgs, ys, proj_weight):
    b, dim, h, w = imgs.shape
    ys32 = ys.astype(jnp.int32)
    imgs_t = imgs.transpose(0, 2, 3, 1)
    out_t = _film_all(imgs_t, ys32, proj_weight)
    return out_t.transpose(0, 3, 1, 2)


# speedup vs baseline: 10.9732x; 1.0039x over previous
import functools

import jax
import jax.numpy as jnp
from jax import lax
from jax.experimental import pallas as pl
from jax.experimental.pallas import tpu as pltpu
from jax.experimental.pallas import tpu_sc as plsc

_NW = 8


def _sc_gather(table, idx):
    n, emb_dim = table.shape
    b = idx.shape[0]
    mesh = plsc.ScalarSubcoreMesh(axis_name="c", num_cores=1)

    @functools.partial(
        pl.kernel,
        mesh=mesh,
        out_type=jax.ShapeDtypeStruct((b, emb_dim), jnp.float32),
        scratch_types=[
            pltpu.SMEM((b,), jnp.int32),
            pltpu.SemaphoreType.DMA,
        ],
    )
    def gather_kernel(table_hbm, idx_hbm, out_hbm, idx_s, sem):
        pltpu.sync_copy(idx_hbm, idx_s)
        copies = [
            pltpu.make_async_copy(
                table_hbm.at[pl.ds(idx_s[i], 1)], out_hbm.at[pl.ds(i, 1)], sem
            )
            for i in range(b)
        ]
        for c in copies:
            c.start()
        for c in copies:
            c.wait()

    return gather_kernel(table, idx)


_BB = 16
_DB = 128
_HEAD = 16
_H1 = 8
_HBB = 4


def _make_film_head_body(start):
    def body(ys_ref, pw_ref, x_ref, *rest):
        o_ref = rest[-1]
        i = pl.program_id(0)
        dim = x_ref.shape[-1]
        rows = [
            pw_ref[pl.ds(ys_ref[start + i * _HBB + k], 1), :]
            for k in range(_HBB)
        ]
        wb = jnp.concatenate(rows, axis=0)
        w = wb[:, :dim]
        bias = wb[:, dim:]
        o_ref[...] = x_ref[...] * w[:, None, None, :] + bias[:, None, None, :]

    return body


def _tc_film_head(imgs_t, ys, table, start, count, partial_out=None):
    b, h, w, dim = imgs_t.shape
    n, emb_dim = table.shape
    i0 = start // _HBB
    in_specs = [
        pl.BlockSpec(memory_space=pltpu.SMEM),
        pl.BlockSpec((n, emb_dim), lambda i: (0, 0)),
        pl.BlockSpec((_HBB, h, w, dim), lambda i: (i + i0, 0, 0, 0)),
    ]
    args = [ys, table, imgs_t]
    aliases = {}
    if partial_out is not None:
        in_specs.append(pl.BlockSpec(memory_space=pl.ANY))
        args.append(partial_out)
        aliases = {3: 0}
    return pl.pallas_call(
        _make_film_head_body(start),
        grid=(count // _HBB,),
        in_specs=in_specs,
        out_specs=pl.BlockSpec((_HBB, h, w, dim), lambda i: (i + i0, 0, 0, 0)),
        out_shape=jax.ShapeDtypeStruct((b, h, w, dim), jnp.float32),
        input_output_aliases=aliases,
    )(*args)


def _film_main_body(w_ref, b_ref, x_ref, part_ref, o_ref):
    del part_ref
    o_ref[...] = (
        x_ref[...] * w_ref[...][:, None, None, :] + b_ref[...][:, None, None, :]
    )


def _film_all(imgs_t, ys32, proj_weight):
    part1 = _tc_film_head(imgs_t, ys32, proj_weight, 0, _H1)
    ys_dep = lax.optimization_barrier((ys32, part1))[0]
    emb = _sc_gather(proj_weight, ys_dep)
    part2 = _tc_film_head(
        imgs_t, ys32, proj_weight, _H1, _HEAD - _H1, partial_out=part1
    )
    return _tc_film_main(imgs_t, emb, part2)


def _tc_film_main(imgs_t, emb, partial_out):
    b, h, w, dim = imgs_t.shape
    nj = dim // _DB
    i0 = _HEAD // _BB
    return pl.pallas_call(
        _film_main_body,
        grid=((b - _HEAD) // _BB, nj),
        in_specs=[
            pl.BlockSpec((_BB, _DB), lambda i, j: (i + i0, j)),
            pl.BlockSpec((_BB, _DB), lambda i, j: (i + i0, j + nj)),
            pl.BlockSpec((_BB, h, w, _DB), lambda i, j: (i + i0, 0, 0, j)),
            pl.BlockSpec(memory_space=pl.ANY),
        ],
        out_specs=pl.BlockSpec((_BB, h, w, _DB), lambda i, j: (i + i0, 0, 0, j)),
        out_shape=jax.ShapeDtypeStruct((b, h, w, dim), jnp.float32),
        input_output_aliases={3: 0},
    )(emb, emb, imgs_t, partial_out)


@jax.jit
def kernel(imgs, ys, proj_weight):
    b, dim, h, w = imgs.shape
    ys32 = ys.astype(jnp.int32)
    imgs_t = imgs.transpose(0, 2, 3, 1)
    out_t = _film_all(imgs_t, ys32, proj_weight)
    return out_t.transpose(0, 3, 1, 2)
